# baseline (speedup 1.0000x reference)
import functools

import jax
import jax.numpy as jnp
from jax import lax
from jax.experimental import pallas as pl
from jax.experimental.pallas import tpu as pltpu
from jax.experimental.pallas import tpu_sc as plsc

B, N, C = 8, 4096, 64
K = 2048
L = 16
NC, NS = 2, 16
NW = NC * NS
Q = NW // B
KQ = K // Q
G = 8
NGROUPS = KQ // G


def _rank_body(s_ref, r_ref):
    v = s_ref[0, 0, :]

    def chunk(c, carry):
        vi = s_ref[0, 0, pl.ds(c * 128, 128)]
        gt = v[None, :] > vi[:, None]
        eq = v[None, :] == vi[:, None]
        jidx = lax.broadcasted_iota(jnp.int32, (128, N), 1)
        iidx = c * 128 + lax.broadcasted_iota(jnp.int32, (128, N), 0)
        contrib = gt | (eq & (jidx < iidx))
        cf = jnp.where(contrib, 1.0, 0.0)
        ones = jnp.ones((N, 8), jnp.float32)
        cnt = jax.lax.dot_general(
            cf, ones, (((1,), (0,)), ((), ())),
            preferred_element_type=jnp.float32)
        r_ref[0, 0, pl.ds(c * 128, 128)] = cnt[:, 0].astype(jnp.int32)
        return carry

    lax.fori_loop(0, N // 128, chunk, 0)


def _compute_ranks(scores):
    out = pl.pallas_call(
        _rank_body,
        grid=(B,),
        in_specs=[pl.BlockSpec((1, 1, N), lambda b: (b, 0, 0))],
        out_specs=pl.BlockSpec((1, 1, N), lambda b: (b, 0, 0)),
        out_shape=jax.ShapeDtypeStruct((B, 1, N), jnp.int32),
    )(scores.reshape(B, 1, N))
    return out.reshape(B, N)


def _wid_bq():
    wid = lax.axis_index("s") * NC + lax.axis_index("c")
    return wid // Q, wid % Q


def _scatter_topk(ranks_v, idx_flat, extra=None):
    def scat(c, carry):
        r_v = ranks_v[pl.ds(c * L, L)]
        i_v = c * L + lax.iota(jnp.int32, L)
        m = r_v < K
        plsc.store_scatter(idx_flat, [r_v], i_v, mask=m)
        for src_ref, dst in extra or ():
            if dst.shape == (K,):
                plsc.store_scatter(dst, [r_v], src_ref[pl.ds(c * L, L)], mask=m)
            else:
                plsc.store_scatter(
                    dst,
                    [lax.shift_right_logical(r_v, 3), lax.bitwise_and(r_v, 7)],
                    i_v, mask=m)
        return carry

    lax.fori_loop(0, N // L, scat, 0)


_SC_PARAMS = dict(
    compiler_params=pltpu.CompilerParams(
        needs_layout_passes=False, use_tc_tiling_on_sc=False),
)


def _mesh():
    return plsc.VectorSubcoreMesh(
        core_axis_name="c", subcore_axis_name="s", num_cores=NC,
        num_subcores=NS)


@functools.cache
def _make_sc_adj(nb=B):
    return functools.partial(
        pl.kernel,
        out_type=(
            jax.ShapeDtypeStruct((nb, K), jnp.int32),
            jax.ShapeDtypeStruct((nb, K, K), jnp.float32),
            jax.ShapeDtypeStruct((nb, K), jnp.float32),
        ),
        mesh=_mesh(),
        scratch_types=[
            pltpu.VMEM((N,), jnp.int32),
            pltpu.VMEM((N,), jnp.float32),
            pltpu.VMEM((K,), jnp.int32),
            pltpu.VMEM((K,), jnp.float32),
            pltpu.VMEM((K // G, G), jnp.int32),
            pltpu.VMEM((G, N), jnp.float32),
            pltpu.VMEM((G, N), jnp.float32),
            pltpu.VMEM((G, K), jnp.float32),
            pltpu.VMEM((G, K), jnp.float32),
            pltpu.SemaphoreType.DMA,
            pltpu.SemaphoreType.DMA,
            pltpu.SemaphoreType.DMA,
            pltpu.SemaphoreType.DMA,
        ],
        **_SC_PARAMS,
    )(functools.partial(_sc_adj_body, nb))


def _sc_adj_body(nb, ranks_hbm, scores_hbm, adj_hbm,
                 topkidx_hbm, seladj_hbm, topkscore_hbm,
                 ranks_v, scores_v, idx_flat, score_buf, idx_dma,
                 rows_a, rows_b, out_a, out_b,
                 sem_ina, sem_inb, sem_outa, sem_outb):
    nq = NW // nb
    kq = K // nq
    ngroups = kq // G
    wid = lax.axis_index("s") * NC + lax.axis_index("c")
    b = wid // nq
    q = wid % nq
    base = q * kq

    pltpu.sync_copy(ranks_hbm.at[b], ranks_v)
    pltpu.sync_copy(scores_hbm.at[b], scores_v)
    _scatter_topk(ranks_v, idx_flat,
                  extra=[(scores_v, score_buf), (None, idx_dma)])

    @pl.when(q == 0)
    def _():
        pltpu.sync_copy(idx_flat, topkidx_hbm.at[b])
        pltpu.sync_copy(score_buf, topkscore_hbm.at[b])

    def start_in(g, rows, sem):
        return pltpu.async_copy(
            adj_hbm.at[idx_dma.at[q * ngroups + g]], rows, sem)

    bufs = ((rows_a, out_a, sem_ina, sem_outa),
            (rows_b, out_b, sem_inb, sem_outb))
    in_d = [start_in(0, rows_a, sem_ina), start_in(1, rows_b, sem_inb)]

    out_d = [None, None]
    for g in range(ngroups):
        p = g & 1
        rows, out, s_in, s_out = bufs[p]
        in_d[p].wait()
        if out_d[p] is not None:
            out_d[p].wait()

        def colchunk(c, inner, rows=rows, out=out):
            idx_v = idx_flat[pl.ds(c * L, L)]
            for r in range(G):
                r_v = jnp.full((L,), r, jnp.int32)
                out[r, pl.ds(c * L, L)] = plsc.load_gather(rows, [r_v, idx_v])
            return inner

        lax.fori_loop(0, K // L, colchunk, 0)
        out_d[p] = pltpu.async_copy(
            out, seladj_hbm.at[b, pl.ds(base + g * G, G)], s_out)
        if g + 2 < ngroups:
            in_d[p] = start_in(g + 2, rows, s_in)
    out_d[0].wait()
    out_d[1].wait()


def _feat_body(r_ref, f_ref, o_ref):
    rankv = r_ref[0, 0, :]

    def rblk(rb, carry):
        rows = rb * 128 + lax.broadcasted_iota(jnp.int32, (128, N), 0)
        p = jnp.where(rankv[None, :] == rows, 1.0, 0.0)
        o_ref[0, pl.ds(rb * 128, 128), :] = jax.lax.dot_general(
            p, f_ref[0], (((1,), (0,)), ((), ())),
            preferred_element_type=jnp.float32)
        return carry

    lax.fori_loop(0, K // 128, rblk, 0)


def _gather_feat_onehot(ranks3d, features):
    return pl.pallas_call(
        _feat_body,
        grid=(B,),
        in_specs=[
            pl.BlockSpec((1, 1, N), lambda b: (b, 0, 0)),
            pl.BlockSpec((1, N, C), lambda b: (b, 0, 0)),
        ],
        out_specs=pl.BlockSpec((1, K, C), lambda b: (b, 0, 0)),
        out_shape=jax.ShapeDtypeStruct((B, K, C), jnp.float32),
    )(ranks3d, features)


def kernel(importance_scores, features, adjacency_matrix):
    ranks3d = pl.pallas_call(
        _rank_body,
        grid=(B,),
        in_specs=[pl.BlockSpec((1, 1, N), lambda b: (b, 0, 0))],
        out_specs=pl.BlockSpec((1, 1, N), lambda b: (b, 0, 0)),
        out_shape=jax.ShapeDtypeStruct((B, 1, N), jnp.int32),
    )(importance_scores.reshape(B, 1, N))
    ranks = ranks3d.reshape(B, N)
    topk_idx, sel_adj, topk_score = _make_sc_adj()(
        ranks, importance_scores, adjacency_matrix)
    sel_feat = _gather_feat_onehot(ranks3d, features)
    return sel_feat, topk_idx, sel_adj, topk_score


# speedup vs baseline: 1.0100x; 1.0100x over previous
import functools

import jax
import jax.numpy as jnp
from jax import lax
from jax.experimental import pallas as pl
from jax.experimental.pallas import tpu as pltpu
from jax.experimental.pallas import tpu_sc as plsc

B, N, C = 8, 4096, 64
K = 2048
L = 16
NC, NS = 2, 16
NW = NC * NS
Q = NW // B
KQ = K // Q
G = 8
NGROUPS = KQ // G


def _rank_body(s_ref, r_ref):
    v = s_ref[0, 0, :]

    def chunk(c, carry):
        vi = s_ref[0, 0, pl.ds(c * 128, 128)]
        gt = v[None, :] > vi[:, None]
        eq = v[None, :] == vi[:, None]
        jidx = lax.broadcasted_iota(jnp.int32, (128, N), 1)
        iidx = c * 128 + lax.broadcasted_iota(jnp.int32, (128, N), 0)
        contrib = gt | (eq & (jidx < iidx))
        cnt = jnp.sum(jnp.where(contrib, 1, 0), axis=1)
        r_ref[0, 0, pl.ds(c * 128, 128)] = cnt
        return carry

    lax.fori_loop(0, N // 128, chunk, 0)


def _compute_ranks(scores):
    out = pl.pallas_call(
        _rank_body,
        grid=(B,),
        in_specs=[pl.BlockSpec((1, 1, N), lambda b: (b, 0, 0))],
        out_specs=pl.BlockSpec((1, 1, N), lambda b: (b, 0, 0)),
        out_shape=jax.ShapeDtypeStruct((B, 1, N), jnp.int32),
    )(scores.reshape(B, 1, N))
    return out.reshape(B, N)


def _wid_bq():
    wid = lax.axis_index("s") * NC + lax.axis_index("c")
    return wid // Q, wid % Q


def _scatter_topk(ranks_v, idx_flat, extra=None):
    def scat(c, carry):
        r_v = ranks_v[pl.ds(c * L, L)]
        i_v = c * L + lax.iota(jnp.int32, L)
        m = r_v < K
        plsc.store_scatter(idx_flat, [r_v], i_v, mask=m)
        for src_ref, dst in extra or ():
            if dst.shape == (K,):
                plsc.store_scatter(dst, [r_v], src_ref[pl.ds(c * L, L)], mask=m)
            else:
                plsc.store_scatter(
                    dst,
                    [lax.shift_right_logical(r_v, 3), lax.bitwise_and(r_v, 7)],
                    i_v, mask=m)
        return carry

    lax.fori_loop(0, N // L, scat, 0)


_SC_PARAMS = dict(
    compiler_params=pltpu.CompilerParams(
        needs_layout_passes=False, use_tc_tiling_on_sc=False),
)


def _mesh():
    return plsc.VectorSubcoreMesh(
        core_axis_name="c", subcore_axis_name="s", num_cores=NC,
        num_subcores=NS)


@functools.cache
def _make_sc_adj(nb=B):
    return functools.partial(
        pl.kernel,
        out_type=(
            jax.ShapeDtypeStruct((nb, K), jnp.int32),
            jax.ShapeDtypeStruct((nb, K, K), jnp.float32),
            jax.ShapeDtypeStruct((nb, K), jnp.float32),
        ),
        mesh=_mesh(),
        scratch_types=[
            pltpu.VMEM((N,), jnp.int32),
            pltpu.VMEM((N,), jnp.float32),
            pltpu.VMEM((K,), jnp.int32),
            pltpu.VMEM((K,), jnp.float32),
            pltpu.VMEM((K // G, G), jnp.int32),
            pltpu.VMEM((G, N), jnp.float32),
            pltpu.VMEM((G, N), jnp.float32),
            pltpu.VMEM((G, K), jnp.float32),
            pltpu.VMEM((G, K), jnp.float32),
            pltpu.SemaphoreType.DMA,
            pltpu.SemaphoreType.DMA,
            pltpu.SemaphoreType.DMA,
            pltpu.SemaphoreType.DMA,
        ],
        **_SC_PARAMS,
    )(functools.partial(_sc_adj_body, nb))


def _sc_adj_body(nb, ranks_hbm, scores_hbm, adj_hbm,
                 topkidx_hbm, seladj_hbm, topkscore_hbm,
                 ranks_v, scores_v, idx_flat, score_buf, idx_dma,
                 rows_a, rows_b, out_a, out_b,
                 sem_ina, sem_inb, sem_outa, sem_outb):
    nq = NW // nb
    kq = K // nq
    ngroups = kq // G
    wid = lax.axis_index("s") * NC + lax.axis_index("c")
    b = wid // nq
    q = wid % nq
    base = q * kq

    pltpu.sync_copy(ranks_hbm.at[b], ranks_v)
    pltpu.sync_copy(scores_hbm.at[b], scores_v)
    _scatter_topk(ranks_v, idx_flat,
                  extra=[(scores_v, score_buf), (None, idx_dma)])

    @pl.when(q == 0)
    def _():
        pltpu.sync_copy(idx_flat, topkidx_hbm.at[b])
        pltpu.sync_copy(score_buf, topkscore_hbm.at[b])

    def start_in(g, rows, sem):
        return pltpu.async_copy(
            adj_hbm.at[idx_dma.at[q * ngroups + g]], rows, sem)

    bufs = ((rows_a, out_a, sem_ina, sem_outa),
            (rows_b, out_b, sem_inb, sem_outb))
    in_d = [start_in(0, rows_a, sem_ina), start_in(1, rows_b, sem_inb)]

    out_d = [None, None]
    for g in range(ngroups):
        p = g & 1
        rows, out, s_in, s_out = bufs[p]
        in_d[p].wait()
        if out_d[p] is not None:
            out_d[p].wait()

        def colchunk(c, inner, rows=rows, out=out):
            idx_v = idx_flat[pl.ds(c * L, L)]
            for r in range(G):
                r_v = jnp.full((L,), r, jnp.int32)
                out[r, pl.ds(c * L, L)] = plsc.load_gather(rows, [r_v, idx_v])
            return inner

        lax.fori_loop(0, K // L, colchunk, 0)
        out_d[p] = pltpu.async_copy(
            out, seladj_hbm.at[b, pl.ds(base + g * G, G)], s_out)
        if g + 2 < ngroups:
            in_d[p] = start_in(g + 2, rows, s_in)
    out_d[0].wait()
    out_d[1].wait()


def _feat_body(r_ref, f_ref, o_ref):
    rankv = r_ref[0, 0, :]

    def rblk(rb, carry):
        rows = rb * 128 + lax.broadcasted_iota(jnp.int32, (128, N), 0)
        p = jnp.where(rankv[None, :] == rows, 1.0, 0.0)
        o_ref[0, pl.ds(rb * 128, 128), :] = jax.lax.dot_general(
            p, f_ref[0], (((1,), (0,)), ((), ())),
            preferred_element_type=jnp.float32)
        return carry

    lax.fori_loop(0, K // 128, rblk, 0)


def _gather_feat_onehot(ranks3d, features):
    return pl.pallas_call(
        _feat_body,
        grid=(B,),
        in_specs=[
            pl.BlockSpec((1, 1, N), lambda b: (b, 0, 0)),
            pl.BlockSpec((1, N, C), lambda b: (b, 0, 0)),
        ],
        out_specs=pl.BlockSpec((1, K, C), lambda b: (b, 0, 0)),
        out_shape=jax.ShapeDtypeStruct((B, K, C), jnp.float32),
    )(ranks3d, features)


def kernel(importance_scores, features, adjacency_matrix):
    ranks3d = pl.pallas_call(
        _rank_body,
        grid=(B,),
        in_specs=[pl.BlockSpec((1, 1, N), lambda b: (b, 0, 0))],
        out_specs=pl.BlockSpec((1, 1, N), lambda b: (b, 0, 0)),
        out_shape=jax.ShapeDtypeStruct((B, 1, N), jnp.int32),
    )(importance_scores.reshape(B, 1, N))
    ranks = ranks3d.reshape(B, N)
    topk_idx, sel_adj, topk_score = _make_sc_adj()(
        ranks, importance_scores, adjacency_matrix)
    sel_feat = _gather_feat_onehot(ranks3d, features)
    return sel_feat, topk_idx, sel_adj, topk_score
